# baseline (speedup 1.0000x reference)
import functools
import jax
import jax.numpy as jnp
from jax import lax
from jax.experimental import pallas as pl
from jax.experimental.pallas import tpu as pltpu
from jax.experimental.pallas import tpu_sc as plsc

_NC = 2
_NS = 16
_NW = _NC * _NS
_C = 8
_NB = 3


def kernel(x, pos_table):
    batch, seq, d = x.shape
    per_w = seq // _NW
    n_chunks = per_w // _C
    n_main = (n_chunks // _NB) * _NB
    mesh = plsc.VectorSubcoreMesh(core_axis_name="c", subcore_axis_name="s")

    @functools.partial(
        pl.kernel,
        out_type=jax.ShapeDtypeStruct((batch, seq, d), jnp.float32),
        mesh=mesh,
        scratch_types=[
            pltpu.VMEM((_NB, _C, d), jnp.float32),
            pltpu.VMEM((_NB, batch, _C, d), jnp.float32),
            pltpu.SemaphoreType.DMA,
            pltpu.SemaphoreType.DMA,
        ],
        compiler_params=pltpu.CompilerParams(use_tc_tiling_on_sc=True),
    )
    def sc_add(x_hbm, pos_hbm, out_hbm, pos_v, x_v, sem_in, sem_out):
        wid = lax.axis_index("s") * _NC + lax.axis_index("c")
        base = wid * per_w

        def issue_in(ci, sl):
            s0 = base + ci * _C
            pltpu.async_copy(pos_hbm.at[pl.ds(s0, _C), :], pos_v.at[sl], sem_in)
            pltpu.async_copy(x_hbm.at[:, pl.ds(s0, _C), :], x_v.at[sl], sem_in)

        def wait_in(sl):
            pltpu.make_async_copy(
                pos_hbm.at[pl.ds(0, _C), :], pos_v.at[sl], sem_in
            ).wait()
            pltpu.make_async_copy(
                x_hbm.at[:, pl.ds(0, _C), :], x_v.at[sl], sem_in
            ).wait()

        def issue_out(ci, sl):
            s0 = base + ci * _C
            pltpu.async_copy(x_v.at[sl], out_hbm.at[:, pl.ds(s0, _C), :], sem_out)

        def wait_out(sl):
            pltpu.make_async_copy(
                x_v.at[sl], out_hbm.at[:, pl.ds(0, _C), :], sem_out
            ).wait()

        def compute(sl):
            for s in range(_C):
                def _add(i, carry, s=s):
                    hsl = pl.ds(i * 16, 16)
                    pv = pos_v[sl, s, hsl]
                    for b in range(batch):
                        x_v[sl, b, s, hsl] = x_v[sl, b, s, hsl] + pv
                    return carry

                lax.fori_loop(0, d // 16, _add, 0, unroll=8)

        issue_in(0, 0)
        issue_in(1, 1)

        @pl.loop(0, n_main, step=_NB)
        def _chunks(ci0):
            for j in range(_NB):
                ci = ci0 + j
                nsl = (j + 2) % _NB

                @pl.when(ci >= 1)
                def _():
                    wait_out(nsl)

                @pl.when(ci + 2 < n_chunks)
                def _():
                    issue_in(ci + 2, nsl)

                wait_in(j)
                compute(j)
                issue_out(ci, j)

        for ci in range(n_main, n_chunks):
            j = ci % _NB
            nsl = (j + 2) % _NB
            wait_out(nsl)

            @pl.when(ci + 2 < n_chunks)
            def _():
                issue_in(ci + 2, nsl)

            wait_in(j)
            compute(j)
            issue_out(ci, j)

        wait_out((n_chunks - 1) % _NB)

    return sc_add(x, pos_table)
